# baseline (speedup 1.0000x reference)
import functools

import jax
import jax.numpy as jnp
from jax import lax
from jax.experimental import pallas as pl
from jax.experimental.pallas import tpu as pltpu
from jax.experimental.pallas import tpu_sc as plsc

N = 100000
HIDDEN = 256
NUM_EMB = 95

NC = 2
NS = 16
NW = NC * NS

CHUNK = 80
NCHUNKS = N // CHUNK


def _swish_table(w):
    def body(w_ref, o_ref):
        v = w_ref[...]
        o_ref[...] = v * (1.0 / (1.0 + jnp.exp(-v)))
    return pl.pallas_call(
        body,
        out_shape=jax.ShapeDtypeStruct(w.shape, w.dtype),
    )(w)


NMAX = -(-NCHUNKS // NW)
NFULL = NCHUNKS - NW * (NMAX - 1)
NBUF = 4


def _make_sc_gather():
    mesh = plsc.VectorSubcoreMesh(core_axis_name="c", subcore_axis_name="s")

    @functools.partial(
        pl.kernel,
        mesh=mesh,
        out_type=jax.ShapeDtypeStruct((N, HIDDEN), jnp.float32),
        scratch_types=[
            pltpu.VMEM((NMAX * CHUNK,), jnp.int32),
            pltpu.VMEM((NBUF, CHUNK, HIDDEN), jnp.float32),
            pltpu.SemaphoreType.DMA,
        ] + [pltpu.SemaphoreType.DMA] * NBUF
          + [pltpu.SemaphoreType.DMA] * NBUF,
    )
    def sc_gather(table_hbm, idx_hbm, out_hbm, idx_all, rows, isem, *sems):
        gsems = sems[:NBUF]
        wsems = sems[NBUF:]
        w = lax.axis_index("s") * NC + lax.axis_index("c")
        last = w < NFULL

        def base_of(i):
            return pl.multiple_of((w + i * NW) * CHUNK, CHUNK)

        def guard(i):
            return None if i < NMAX - 1 else last

        def run(g, fn):
            if g is None:
                fn()
            else:
                pl.when(g)(fn)

        icps = [
            pltpu.make_async_copy(
                idx_hbm.at[pl.ds(base_of(i), CHUNK)],
                idx_all.at[pl.ds(i * CHUNK, CHUNK)],
                isem,
            )
            for i in range(NMAX)
        ]
        for i in range(NMAX):
            run(guard(i), icps[i].start)
        for i in range(NMAX):
            run(guard(i), icps[i].wait)

        gcps = [
            pltpu.make_async_copy(
                table_hbm.at[idx_all.at[pl.ds(i * CHUNK, CHUNK)]],
                rows.at[i % NBUF],
                gsems[i % NBUF],
            )
            for i in range(NMAX)
        ]
        wcps = [
            pltpu.make_async_copy(
                rows.at[i % NBUF],
                out_hbm.at[pl.ds(base_of(i), CHUNK)],
                wsems[i % NBUF],
            )
            for i in range(NMAX)
        ]
        for i in range(NMAX):
            if i >= NBUF:
                run(guard(i - NBUF), wcps[i - NBUF].wait)
            run(guard(i), gcps[i].start)
            if i >= 1:
                run(guard(i - 1), gcps[i - 1].wait)
                run(guard(i - 1), wcps[i - 1].start)
        run(guard(NMAX - 1), gcps[NMAX - 1].wait)
        run(guard(NMAX - 1), wcps[NMAX - 1].start)
        for i in range(max(0, NMAX - NBUF), NMAX):
            run(guard(i), wcps[i].wait)

    return sc_gather


_sc_gather = _make_sc_gather()


def kernel(x, emb_weight):
    act_table = _swish_table(emb_weight)
    return _sc_gather(act_table, x.astype(jnp.int32))
